# Initial kernel scaffold; baseline (speedup 1.0000x reference)
import jax
import jax.numpy as jnp
from jax.experimental import pallas as pl


def kernel(x, W):
    raise NotImplementedError("write your pallas kernel here")


# speedup vs baseline: 1.2730x; 1.2730x over previous
import functools

import jax
import jax.numpy as jnp
from jax.experimental import pallas as pl
from jax.experimental.pallas import tpu as pltpu

NUM_ROWS = 100000
DIM = 4096
TOPK = 32

BLOCK_ROWS = 1024
NUM_BLOCKS = 98
PAD_ROWS = NUM_BLOCKS * BLOCK_ROWS
SUBS = PAD_ROWS // 128

NEG_INF = float("-inf")


def _fused_kernel(x_ref, w_ref, out_ref, vals_ref, inds_ref):
    i = pl.program_id(0)

    y = jax.lax.dot_general(
        w_ref[...].astype(jnp.bfloat16), x_ref[...].astype(jnp.bfloat16),
        dimension_numbers=(((1,), (0,)), ((), ())),
        preferred_element_type=jnp.float32,
    )
    y2 = y.reshape(8, 128)
    sub = jax.lax.broadcasted_iota(jnp.int32, (8, 128), 0)
    lane = jax.lax.broadcasted_iota(jnp.int32, (8, 128), 1)
    g = i * BLOCK_ROWS + sub * 128 + lane
    y2 = jnp.where(g < NUM_ROWS, y2, NEG_INF)
    out_ref[pl.ds(i * 8, 8), :] = y2

    @pl.when(i == NUM_BLOCKS - 1)
    def _():
        bsub = jax.lax.broadcasted_iota(jnp.int32, (SUBS, 128), 0)
        blane = jax.lax.broadcasted_iota(jnp.int32, (SUBS, 128), 1)
        bi = bsub * 128 + blane

        def body(t, _):
            s = out_ref[...]
            m = jnp.max(s)
            idx = jnp.min(jnp.where(s == m, bi, jnp.int32(2**30)))
            vals_ref[t] = m
            inds_ref[t] = idx
            out_ref[...] = jnp.where(bi == idx, NEG_INF, s)
            return 0

        jax.lax.fori_loop(0, TOPK + 1, body, 0, unroll=False)

        v0 = vals_ref[0]
        vlast = vals_ref[TOPK]
        inv = 1.0 / (v0 - vlast)
        out_ref[...] = jnp.zeros((SUBS, 128), jnp.float32)

        def sbody(t, _):
            sv = (vals_ref[t] - vlast) * inv
            out_ref[...] = jnp.where(bi == inds_ref[t], sv, out_ref[...])
            return 0

        jax.lax.fori_loop(0, TOPK, sbody, 0, unroll=False)


@jax.jit
def kernel(x, W):
    res = pl.pallas_call(
        _fused_kernel,
        grid=(NUM_BLOCKS,),
        in_specs=[
            pl.BlockSpec((DIM,), lambda i: (0,)),
            pl.BlockSpec((BLOCK_ROWS, DIM), lambda i: (i, 0)),
        ],
        out_specs=pl.BlockSpec((SUBS, 128), lambda i: (0, 0)),
        out_shape=jax.ShapeDtypeStruct((SUBS, 128), jnp.float32),
        scratch_shapes=[
            pltpu.SMEM((TOPK + 1,), jnp.float32),
            pltpu.SMEM((TOPK + 1,), jnp.int32),
        ],
    )(x, W)
    return res.reshape(PAD_ROWS)[:NUM_ROWS]
